# baseline (speedup 1.0000x reference)
import functools

import jax
import jax.numpy as jnp
from jax import lax
from jax.experimental import pallas as pl
from jax.experimental.pallas import tpu as pltpu
from jax.experimental.pallas import tpu_sc as plsc

H = 128
T = 200
CHUNK = 8
L = 16
NC, NS = 2, 16
NW = NC * NS
GCHUNKS = ((0, 0, 128), (1, 128, 72))
TPAD = 256
HV = H // L


def _pool_body(ids_hbm, table_hbm, out_hbm, idx_v, rows_v, acc_v, sem0, sem1, sem2):
    rpb = TPAD // 128
    bpw = ids_hbm.shape[0] // rpb // NW
    wid = lax.axis_index("s") * NC + lax.axis_index("c")
    base = wid * bpw

    pltpu.sync_copy(ids_hbm.at[pl.ds(base * rpb, bpw * rpb)], idx_v)

    def fire(b, buf, sem):
        for row, off, ln in GCHUNKS:
            pltpu.async_copy(
                table_hbm.at[idx_v.at[b * rpb + row, pl.ds(0, ln)]],
                rows_v.at[buf, pl.ds(off, ln)],
                sem,
            )

    def drain(buf, sem):
        for _, off, ln in GCHUNKS:
            pltpu.make_async_copy(
                table_hbm.at[pl.ds(0, ln)],
                rows_v.at[buf, pl.ds(off, ln)],
                sem,
            ).wait()

    def accum(b, buf):
        def body(t, accs):
            return tuple(
                accs[h] + rows_v[buf, t, pl.ds(h * L, L)] for h in range(HV)
            )
        accs = tuple(jnp.zeros((L,), jnp.float32) for _ in range(HV))
        accs = tuple(a + rows_v[buf, 0, pl.ds(h * L, L)] for h, a in enumerate(accs))
        for h in range(HV):
            acc_v[b, pl.ds(h * L, L)] = accs[h] * (1.0 / T)

    fire(0, 0, sem0)
    fire(1, 1, sem1)

    def step(i, _):
        b0 = 3 * i
        fire(b0 + 2, 2, sem2)
        drain(0, sem0)
        accum(b0, 0)
        fire(b0 + 3, 0, sem0)
        drain(1, sem1)
        accum(b0 + 1, 1)
        fire(b0 + 4, 1, sem1)
        drain(2, sem2)
        accum(b0 + 2, 2)
        return 0

    lax.fori_loop(0, (bpw - 2) // 3, step, 0)
    drain(0, sem0)
    accum(bpw - 2, 0)
    drain(1, sem1)
    accum(bpw - 1, 1)

    pltpu.sync_copy(acc_v, out_hbm.at[pl.ds(base, bpw)])


def _pooled(ids_sc, emb_table):
    batch = ids_sc.shape[0] * 128 // TPAD
    bpw = batch // NW
    mesh = plsc.VectorSubcoreMesh(
        core_axis_name="c", subcore_axis_name="s", num_cores=NC, num_subcores=NS
    )
    f = functools.partial(
        pl.kernel,
        mesh=mesh,
        compiler_params=pltpu.CompilerParams(use_tc_tiling_on_sc=False),
        out_type=jax.ShapeDtypeStruct((batch, H), jnp.float32),
        scratch_types=[
            pltpu.VMEM((bpw * TPAD // 128, 128), jnp.int32),
            pltpu.VMEM((3, T, H), jnp.float32),
            pltpu.VMEM((bpw, H), jnp.float32),
            pltpu.SemaphoreType.DMA,
            pltpu.SemaphoreType.DMA,
            pltpu.SemaphoreType.DMA,
        ],
    )(_pool_body)
    return f(ids_sc, emb_table)


def _proj_body(x_ref, wt_ref, b_ref, o_ref):
    x = x_ref[...]
    for c in range(CHUNK):
        o_ref[:, c, :] = (
            jnp.dot(x, wt_ref[:, c, :], preferred_element_type=jnp.float32)
            + b_ref[c, :]
        )


def _proj(pooled, wt, bias):
    bm = 512
    batch = pooled.shape[0]
    return pl.pallas_call(
        _proj_body,
        grid=(batch // bm,),
        in_specs=[
            pl.BlockSpec((bm, H), lambda i: (i, 0)),
            pl.BlockSpec((H, CHUNK, H), lambda i: (0, 0, 0)),
            pl.BlockSpec((CHUNK, H), lambda i: (0, 0)),
        ],
        out_specs=pl.BlockSpec((bm, CHUNK, H), lambda i: (i, 0, 0)),
        out_shape=jax.ShapeDtypeStruct((batch, CHUNK, H), jnp.float32),
    )(pooled, wt, bias)


def kernel(thought_ids, emb_table, W, b):
    batch = thought_ids.shape[0]
    wt = W.T.reshape(H, CHUNK, H)
    bias = b.reshape(CHUNK, H)
    ids_sc = jnp.pad(thought_ids, ((0, 0), (0, TPAD - T))).reshape(-1, 128)
    pooled = _pooled(ids_sc, emb_table)
    return _proj(pooled, wt, bias)


# speedup vs baseline: 1.1203x; 1.1203x over previous
import functools

import jax
import jax.numpy as jnp
from jax import lax
from jax.experimental import pallas as pl
from jax.experimental.pallas import tpu as pltpu
from jax.experimental.pallas import tpu_sc as plsc

H = 128
T = 200
CHUNK = 8
L = 16
NC, NS = 2, 16
NW = NC * NS
GCHUNKS = ((0, 0, 128), (1, 128, 72))
TPAD = 256
HV = H // L


def _pool_body(ids_hbm, table_hbm, out_hbm, idx_v, rows_v, acc_v, sem0, sem1, sem2):
    rpb = TPAD // 128
    bpw = ids_hbm.shape[0] // rpb // NW
    wid = lax.axis_index("s") * NC + lax.axis_index("c")
    base = wid * bpw

    pltpu.sync_copy(ids_hbm.at[pl.ds(base * rpb, bpw * rpb)], idx_v)

    def fire(b, buf, sem):
        for row, off, ln in GCHUNKS:
            pltpu.async_copy(
                table_hbm.at[idx_v.at[b * rpb + row, pl.ds(0, ln)]],
                rows_v.at[buf, pl.ds(off, ln)],
                sem,
            )

    def drain(buf, sem):
        for _, off, ln in GCHUNKS:
            pltpu.make_async_copy(
                table_hbm.at[pl.ds(0, ln)],
                rows_v.at[buf, pl.ds(off, ln)],
                sem,
            ).wait()

    def accum(b, buf):
        def body(t, accs):
            return tuple(
                accs[h] + rows_v[buf, t, pl.ds(h * L, L)] for h in range(HV)
            )
        accs = tuple(jnp.zeros((L,), jnp.float32) for _ in range(HV))
        accs = plsc.parallel_loop(0, T, 1, unroll=4, carry=accs)(body)
        for h in range(HV):
            acc_v[b, pl.ds(h * L, L)] = accs[h] * (1.0 / T)

    fire(0, 0, sem0)
    fire(1, 1, sem1)

    def step(i, _):
        b0 = 3 * i
        fire(b0 + 2, 2, sem2)
        drain(0, sem0)
        accum(b0, 0)
        fire(b0 + 3, 0, sem0)
        drain(1, sem1)
        accum(b0 + 1, 1)
        fire(b0 + 4, 1, sem1)
        drain(2, sem2)
        accum(b0 + 2, 2)
        return 0

    lax.fori_loop(0, (bpw - 2) // 3, step, 0)
    drain(0, sem0)
    accum(bpw - 2, 0)
    drain(1, sem1)
    accum(bpw - 1, 1)

    pltpu.sync_copy(acc_v, out_hbm.at[pl.ds(base, bpw)])


def _pooled(ids_sc, emb_table):
    batch = ids_sc.shape[0] * 128 // TPAD
    bpw = batch // NW
    mesh = plsc.VectorSubcoreMesh(
        core_axis_name="c", subcore_axis_name="s", num_cores=NC, num_subcores=NS
    )
    f = functools.partial(
        pl.kernel,
        mesh=mesh,
        compiler_params=pltpu.CompilerParams(use_tc_tiling_on_sc=False),
        out_type=jax.ShapeDtypeStruct((batch, H), jnp.float32),
        scratch_types=[
            pltpu.VMEM((bpw * TPAD // 128, 128), jnp.int32),
            pltpu.VMEM((3, T, H), jnp.float32),
            pltpu.VMEM((bpw, H), jnp.float32),
            pltpu.SemaphoreType.DMA,
            pltpu.SemaphoreType.DMA,
            pltpu.SemaphoreType.DMA,
        ],
    )(_pool_body)
    return f(ids_sc, emb_table)


def _proj_body(x_ref, wt_ref, b_ref, o_ref):
    x = x_ref[...]
    for c in range(CHUNK):
        o_ref[:, c, :] = (
            jnp.dot(x, wt_ref[:, c, :], preferred_element_type=jnp.float32)
            + b_ref[c, :]
        )


def _proj(pooled, wt, bias):
    bm = 512
    batch = pooled.shape[0]
    return pl.pallas_call(
        _proj_body,
        grid=(batch // bm,),
        in_specs=[
            pl.BlockSpec((bm, H), lambda i: (i, 0)),
            pl.BlockSpec((H, CHUNK, H), lambda i: (0, 0, 0)),
            pl.BlockSpec((CHUNK, H), lambda i: (0, 0)),
        ],
        out_specs=pl.BlockSpec((bm, CHUNK, H), lambda i: (i, 0, 0)),
        out_shape=jax.ShapeDtypeStruct((batch, CHUNK, H), jnp.float32),
    )(pooled, wt, bias)


def kernel(thought_ids, emb_table, W, b):
    batch = thought_ids.shape[0]
    wt = W.T.reshape(H, CHUNK, H)
    bias = b.reshape(CHUNK, H)
    ids_sc = jnp.pad(thought_ids, ((0, 0), (0, TPAD - T))).reshape(-1, 128)
    pooled = _pooled(ids_sc, emb_table)
    return _proj(pooled, wt, bias)
